# Initial kernel scaffold; baseline (speedup 1.0000x reference)
import jax
import jax.numpy as jnp
from jax.experimental import pallas as pl


def kernel(x, table, W, b):
    raise NotImplementedError("write your pallas kernel here")


# speedup vs baseline: 55.7987x; 55.7987x over previous
import jax
import jax.numpy as jnp
from jax import lax
from jax.experimental import pallas as pl
from jax.experimental.pallas import tpu as pltpu
from jax.experimental.pallas import tpu_sc as plsc

NUM_ROWS = 16 * 1024 * 1024
DIM = 4
BATCH = 16384
FEATS = 26
B_TOTAL = BATCH * FEATS
NW = 32
B_PER_W = B_TOTAL // NW
CHUNK = 128
NCHUNK = B_PER_W // CHUNK
LANES = 16
GRP = CHUNK // LANES
OUT_W = B_PER_W * DIM


def _sc_body(x_hbm, t_hbm, w_hbm, b_hbm, out_hbm,
             idx_v, gidx_a, gidx_b, col_a, col_b, out_v, w_v, b_v,
             sem_a, sem_b, sem_o):
    c_ax = lax.axis_index("c")
    s_ax = lax.axis_index("s")
    wid = s_ax * 2 + c_ax

    pltpu.sync_copy(x_hbm.at[pl.ds(wid * NCHUNK, NCHUNK)],
                    idx_v.at[pl.ds(0, NCHUNK)])
    pltpu.sync_copy(x_hbm.at[pl.ds(wid * NCHUNK, 1)],
                    idx_v.at[pl.ds(NCHUNK, 1)])
    pltpu.sync_copy(w_hbm, w_v)
    pltpu.sync_copy(b_hbm, b_v)

    w_reg = [[w_v[pl.ds((o * DIM + c) * LANES, LANES)] for c in range(DIM)]
             for o in range(DIM)]
    b_reg = [b_v[pl.ds(o * LANES, LANES)] for o in range(DIM)]
    out_base = wid * OUT_W

    def build_and_fire(j, gidx_v, col_v, sem):
        for g in range(GRP):
            xv = idx_v[j, pl.ds(g * LANES, LANES)]
            base = ((xv >> 7) << 9) + (xv & 127)
            for c in range(DIM):
                gidx_v[c, pl.ds(g * LANES, LANES)] = base + (c * 128)
        for c in range(DIM):
            pltpu.async_copy(t_hbm.at[gidx_v.at[c]], col_v.at[c], sem)

    def wait_gathers(gidx_v, col_v, sem):
        for c in range(DIM):
            pltpu.make_async_copy(t_hbm.at[gidx_v.at[c]], col_v.at[c],
                                  sem).wait()

    def compute(j, col_v):
        for g in range(GRP):
            v = [col_v[c, pl.ds(g * LANES, LANES)] for c in range(DIM)]
            for o in range(DIM):
                acc = b_reg[o]
                for c in range(DIM):
                    acc = acc + w_reg[o][c] * v[c]
                out_v[pl.ds(j * (CHUNK * DIM) + o * CHUNK + g * LANES,
                            LANES)] = acc
        src = out_v.at[pl.ds(pl.multiple_of(j * (CHUNK * DIM), 8),
                             CHUNK * DIM)]
        dst = out_hbm.at[pl.ds(pl.multiple_of(out_base + j * (CHUNK * DIM), 8),
                               CHUNK * DIM)]
        pltpu.async_copy(src, dst, sem_o)

    build_and_fire(0, gidx_a, col_a, sem_a)

    def pair_body(k):
        j = pl.multiple_of(k * 2, 2)
        build_and_fire(j + 1, gidx_b, col_b, sem_b)
        wait_gathers(gidx_a, col_a, sem_a)
        compute(j, col_a)
        build_and_fire(j + 2, gidx_a, col_a, sem_a)
        wait_gathers(gidx_b, col_b, sem_b)
        compute(j + 1, col_b)

    pl.loop(0, NCHUNK // 2)(pair_body)

    wait_gathers(gidx_a, col_a, sem_a)
    pltpu.make_async_copy(out_v, out_hbm.at[pl.ds(out_base, OUT_W)],
                          sem_o).wait()


@jax.jit
def _run(xcols, tnat, wsplat, bsplat):
    mesh = plsc.VectorSubcoreMesh(core_axis_name="c", subcore_axis_name="s")
    f = pl.kernel(
        _sc_body,
        out_type=jax.ShapeDtypeStruct((B_TOTAL * DIM,), jnp.float32),
        mesh=mesh,
        compiler_params=pltpu.CompilerParams(needs_layout_passes=False,
                                             use_tc_tiling_on_sc=False),
        scratch_types=[
            pltpu.VMEM((NCHUNK + 1, CHUNK), jnp.int32),
            pltpu.VMEM((DIM, CHUNK), jnp.int32),
            pltpu.VMEM((DIM, CHUNK), jnp.int32),
            pltpu.VMEM((DIM, CHUNK), jnp.float32),
            pltpu.VMEM((DIM, CHUNK), jnp.float32),
            pltpu.VMEM((OUT_W,), jnp.float32),
            pltpu.VMEM((DIM * DIM * LANES,), jnp.float32),
            pltpu.VMEM((DIM * LANES,), jnp.float32),
            pltpu.SemaphoreType.DMA,
            pltpu.SemaphoreType.DMA,
            pltpu.SemaphoreType.DMA,
        ],
    )
    return f(xcols, tnat, wsplat, bsplat)


def kernel(x, table, W, b):
    xcols = x.astype(jnp.int32).T.reshape(FEATS * (BATCH // CHUNK), CHUNK)
    tnat = table.reshape(NUM_ROWS // CHUNK, CHUNK, DIM)
    tnat = tnat.transpose(0, 2, 1).reshape(NUM_ROWS * DIM)
    wsplat = jnp.repeat(W.reshape(DIM * DIM), LANES)
    bsplat = jnp.repeat(b, LANES)
    out_flat = _run(xcols, tnat, wsplat, bsplat)
    out = out_flat.reshape(FEATS, BATCH // CHUNK, DIM, CHUNK)
    return out.transpose(1, 3, 0, 2).reshape(BATCH, FEATS, DIM)
